# Initial kernel scaffold; baseline (speedup 1.0000x reference)
import jax
import jax.numpy as jnp
from jax.experimental import pallas as pl


def kernel(inputs, gate_kernel, w0_kernel, w1_kernel, wo_kernel):
    raise NotImplementedError("write your pallas kernel here")


# speedup vs baseline: 2.6034x; 2.6034x over previous
import functools

import jax
import jax.numpy as jnp
from jax import lax
from jax.experimental import pallas as pl
from jax.experimental.pallas import tpu as pltpu

NUM_EXPERTS = 8
TOP_K = 2
EMB = 1024
MLP = 4096

TM = 256
P_MAX = ((2048 * TOP_K + NUM_EXPERTS * (TM - 1)) + TM - 1) // TM * TM
U_MAX = P_MAX // TM
TN_UP = 512
TN_DN = 512


def _up_body(meta_ref, x_ref, w0_ref, w1_ref, out_ref):
    x = x_ref[...]
    a0 = jnp.dot(x, w0_ref[0], preferred_element_type=jnp.float32)
    a1 = jnp.dot(x, w1_ref[0], preferred_element_type=jnp.float32)
    out_ref[...] = (a0 * jax.nn.sigmoid(a0)) * a1


def _dn_body(meta_ref, x_ref, wo_ref, out_ref):
    out_ref[...] = jnp.dot(x_ref[...], wo_ref[0],
                           preferred_element_type=jnp.float32)


def _up_gmm(meta, xs, w0, w1):
    grid = (MLP // TN_UP, U_MAX)
    return pl.pallas_call(
        _up_body,
        grid_spec=pltpu.PrefetchScalarGridSpec(
            num_scalar_prefetch=1,
            grid=grid,
            in_specs=[
                pl.BlockSpec((TM, EMB), lambda n, u, m: (u, 0)),
                pl.BlockSpec((1, EMB, TN_UP), lambda n, u, m: (m[u], 0, n)),
                pl.BlockSpec((1, EMB, TN_UP), lambda n, u, m: (m[u], 0, n)),
            ],
            out_specs=pl.BlockSpec((TM, TN_UP), lambda n, u, m: (u, n)),
        ),
        out_shape=jax.ShapeDtypeStruct((P_MAX, MLP), jnp.float32),
    )(meta, xs, w0, w1)


def _dn_gmm(meta, inter, wo):
    grid = (EMB // TN_DN, U_MAX)
    return pl.pallas_call(
        _dn_body,
        grid_spec=pltpu.PrefetchScalarGridSpec(
            num_scalar_prefetch=1,
            grid=grid,
            in_specs=[
                pl.BlockSpec((TM, MLP), lambda n, u, m: (u, 0)),
                pl.BlockSpec((1, MLP, TN_DN), lambda n, u, m: (m[u], 0, n)),
            ],
            out_specs=pl.BlockSpec((TM, TN_DN), lambda n, u, m: (u, n)),
        ),
        out_shape=jax.ShapeDtypeStruct((P_MAX, EMB), jnp.float32),
    )(meta, inter, wo)


def kernel(inputs, gate_kernel, w0_kernel, w1_kernel, wo_kernel):
    inputs = inputs.astype(jnp.float32)
    x2 = inputs.reshape(-1, EMB)
    T = x2.shape[0]

    logits = x2 @ gate_kernel
    top_w, sel = lax.top_k(logits, TOP_K)
    top_w = jax.nn.softmax(top_w, axis=-1)

    flat_e = sel.reshape(-1)
    oh = jax.nn.one_hot(flat_e, NUM_EXPERTS, dtype=jnp.int32)
    counts = jnp.sum(oh, axis=0)
    padded = (counts + TM - 1) // TM * TM
    poff = jnp.concatenate([jnp.zeros((1,), jnp.int32),
                            jnp.cumsum(padded).astype(jnp.int32)])
    rank = jnp.take_along_axis(jnp.cumsum(oh, axis=0) - oh,
                               flat_e[:, None], axis=1)[:, 0]
    pos = poff[flat_e] + rank

    src = jnp.zeros((P_MAX,), jnp.int32).at[pos].set(
        jnp.arange(T * TOP_K, dtype=jnp.int32) // TOP_K)
    xs = jnp.take(x2, src, axis=0)

    block_expert = jnp.clip(
        jnp.searchsorted(poff, jnp.arange(U_MAX, dtype=jnp.int32) * TM,
                         side="right") - 1,
        0, NUM_EXPERTS - 1).astype(jnp.int32)

    inter = _up_gmm(block_expert, xs, w0_kernel, w1_kernel)
    y = _dn_gmm(block_expert, inter, wo_kernel)

    yk = jnp.take(y, pos, axis=0).reshape(T, TOP_K, EMB)
    out = jnp.sum(yk * top_w[:, :, None], axis=1)
    return out.reshape(inputs.shape)


# speedup vs baseline: 2.7846x; 1.0696x over previous
import functools

import jax
import jax.numpy as jnp
from jax import lax
from jax.experimental import pallas as pl
from jax.experimental.pallas import tpu as pltpu

NUM_EXPERTS = 8
TOP_K = 2
EMB = 1024
MLP = 4096

TM = 256
P_MAX = ((2048 * TOP_K + NUM_EXPERTS * (TM - 1)) + TM - 1) // TM * TM
U_MAX = P_MAX // TM
TN_UP = 512
TN_DN = 512


def _up_body(meta_ref, x_ref, w0_ref, w1_ref, out_ref):
    x = x_ref[...]
    a0 = jnp.dot(x, w0_ref[0].astype(jnp.bfloat16),
                 preferred_element_type=jnp.float32)
    a1 = jnp.dot(x, w1_ref[0].astype(jnp.bfloat16),
                 preferred_element_type=jnp.float32)
    out_ref[...] = ((a0 * jax.nn.sigmoid(a0)) * a1).astype(jnp.bfloat16)


def _dn_body(meta_ref, x_ref, wo_ref, out_ref):
    out_ref[...] = jnp.dot(x_ref[...], wo_ref[0].astype(jnp.bfloat16),
                           preferred_element_type=jnp.float32)


def _up_gmm(meta, xs, w0, w1):
    grid = (MLP // TN_UP, U_MAX)
    return pl.pallas_call(
        _up_body,
        grid_spec=pltpu.PrefetchScalarGridSpec(
            num_scalar_prefetch=1,
            grid=grid,
            in_specs=[
                pl.BlockSpec((TM, EMB), lambda n, u, m: (u, 0)),
                pl.BlockSpec((1, EMB, TN_UP), lambda n, u, m: (m[u], 0, n)),
                pl.BlockSpec((1, EMB, TN_UP), lambda n, u, m: (m[u], 0, n)),
            ],
            out_specs=pl.BlockSpec((TM, TN_UP), lambda n, u, m: (u, n)),
        ),
        out_shape=jax.ShapeDtypeStruct((P_MAX, MLP), jnp.bfloat16),
    )(meta, xs, w0, w1)


def _dn_gmm(meta, inter, wo):
    grid = (EMB // TN_DN, U_MAX)
    return pl.pallas_call(
        _dn_body,
        grid_spec=pltpu.PrefetchScalarGridSpec(
            num_scalar_prefetch=1,
            grid=grid,
            in_specs=[
                pl.BlockSpec((TM, MLP), lambda n, u, m: (u, 0)),
                pl.BlockSpec((1, MLP, TN_DN), lambda n, u, m: (m[u], 0, n)),
            ],
            out_specs=pl.BlockSpec((TM, TN_DN), lambda n, u, m: (u, n)),
        ),
        out_shape=jax.ShapeDtypeStruct((P_MAX, EMB), jnp.float32),
    )(meta, inter, wo)


def kernel(inputs, gate_kernel, w0_kernel, w1_kernel, wo_kernel):
    inputs = inputs.astype(jnp.float32)
    x2 = inputs.reshape(-1, EMB)
    T = x2.shape[0]

    logits = x2 @ gate_kernel
    top_w, sel = lax.top_k(logits, TOP_K)
    top_w = jax.nn.softmax(top_w, axis=-1)

    flat_e = sel.reshape(-1)
    oh = jax.nn.one_hot(flat_e, NUM_EXPERTS, dtype=jnp.int32)
    counts = jnp.sum(oh, axis=0)
    padded = (counts + TM - 1) // TM * TM
    poff = jnp.concatenate([jnp.zeros((1,), jnp.int32),
                            jnp.cumsum(padded).astype(jnp.int32)])
    rank = jnp.take_along_axis(jnp.cumsum(oh, axis=0) - oh,
                               flat_e[:, None], axis=1)[:, 0]
    pos = poff[flat_e] + rank

    src = jnp.zeros((P_MAX,), jnp.int32).at[pos].set(
        jnp.arange(T * TOP_K, dtype=jnp.int32) // TOP_K)
    xs = jnp.take(x2.astype(jnp.bfloat16), src, axis=0)

    block_expert = jnp.clip(
        jnp.searchsorted(poff, jnp.arange(U_MAX, dtype=jnp.int32) * TM,
                         side="right") - 1,
        0, NUM_EXPERTS - 1).astype(jnp.int32)

    inter = _up_gmm(block_expert, xs, w0_kernel, w1_kernel)
    y = _dn_gmm(block_expert, inter, wo_kernel)

    yk = jnp.take(y, pos, axis=0).reshape(T, TOP_K, EMB)
    out = jnp.sum(yk * top_w[:, :, None], axis=1)
    return out.reshape(inputs.shape)


# speedup vs baseline: 3.7578x; 1.3495x over previous
import functools

import jax
import jax.numpy as jnp
from jax import lax
from jax.experimental import pallas as pl
from jax.experimental.pallas import tpu as pltpu

NUM_EXPERTS = 8
TOP_K = 2
EMB = 1024
MLP = 4096

TM = 256
P_MAX = ((2048 * TOP_K + NUM_EXPERTS * (TM - 1)) + TM - 1) // TM * TM
U_MAX = P_MAX // TM
TN_UP = 2048
TN_DN = 1024
TB = 512


def _route_body(logits_ref, a1_ref, a2_ref, wa_ref, wb_ref,
                r0_ref, r1_ref, cnt_ref, carry_ref):
    g = pl.program_id(0)

    @pl.when(g == 0)
    def _():
        carry_ref[...] = jnp.zeros_like(carry_ref)

    logits = logits_ref[...]
    idx = lax.broadcasted_iota(jnp.int32, (TB, NUM_EXPERTS), 1)
    m1 = jnp.max(logits, axis=1, keepdims=True)
    a1 = jnp.min(jnp.where(logits == m1, idx, NUM_EXPERTS), axis=1)
    not1 = idx != a1[:, None]
    m2 = jnp.max(jnp.where(not1, logits, -jnp.inf), axis=1, keepdims=True)
    a2 = jnp.min(jnp.where((logits == m2) & not1, idx, NUM_EXPERTS), axis=1)
    wa = jax.nn.sigmoid(m1 - m2)[:, 0]

    oh = ((idx == a1[:, None]) | (idx == a2[:, None])).astype(jnp.float32)
    lt = (lax.broadcasted_iota(jnp.int32, (TB, TB), 0)
          > lax.broadcasted_iota(jnp.int32, (TB, TB), 1)).astype(jnp.bfloat16)
    c_excl = carry_ref[...] + jnp.dot(lt, oh.astype(jnp.bfloat16),
                                      preferred_element_type=jnp.float32)
    r0 = jnp.sum(jnp.where(idx == a1[:, None], c_excl, 0.0), axis=1)
    r1 = jnp.sum(jnp.where(idx == a2[:, None], c_excl, 0.0), axis=1)

    a1_ref[...] = a1.astype(jnp.int32)
    a2_ref[...] = a2.astype(jnp.int32)
    wa_ref[...] = wa
    wb_ref[...] = 1.0 - wa
    r0_ref[...] = r0.astype(jnp.int32)
    r1_ref[...] = r1.astype(jnp.int32)
    carry_ref[...] += jnp.sum(oh, axis=0, keepdims=True)

    @pl.when(g == pl.num_programs(0) - 1)
    def _():
        cnt_ref[...] = carry_ref[0].astype(jnp.int32)


def _route(logits):
    T = logits.shape[0]
    n = T // TB
    vec = lambda d: jax.ShapeDtypeStruct((T,), d)
    return pl.pallas_call(
        _route_body,
        grid=(n,),
        in_specs=[
            pl.BlockSpec((TB, NUM_EXPERTS), lambda g: (g, 0)),
        ],
        out_specs=[pl.BlockSpec((TB,), lambda g: (g,))] * 6
        + [pl.BlockSpec((NUM_EXPERTS,), lambda g: (0,))],
        out_shape=[vec(jnp.int32), vec(jnp.int32), vec(jnp.float32),
                   vec(jnp.float32), vec(jnp.int32), vec(jnp.int32),
                   jax.ShapeDtypeStruct((NUM_EXPERTS,), jnp.int32)],
        scratch_shapes=[pltpu.VMEM((1, NUM_EXPERTS), jnp.float32)],
    )(logits)


def _up_body(meta_ref, x_ref, w0_ref, w1_ref, out_ref):
    x = x_ref[...]
    a0 = jnp.dot(x, w0_ref[0].astype(jnp.bfloat16),
                 preferred_element_type=jnp.float32)
    a1 = jnp.dot(x, w1_ref[0].astype(jnp.bfloat16),
                 preferred_element_type=jnp.float32)
    out_ref[...] = ((a0 * jax.nn.sigmoid(a0)) * a1).astype(jnp.bfloat16)


def _dn_body(meta_ref, x_ref, wo_ref, out_ref):
    out_ref[...] = jnp.dot(x_ref[...], wo_ref[0].astype(jnp.bfloat16),
                           preferred_element_type=jnp.float32)


def _up_gmm(meta, xs, w0, w1):
    grid = (MLP // TN_UP, U_MAX)
    return pl.pallas_call(
        _up_body,
        grid_spec=pltpu.PrefetchScalarGridSpec(
            num_scalar_prefetch=1,
            grid=grid,
            in_specs=[
                pl.BlockSpec((TM, EMB), lambda n, u, m: (u, 0)),
                pl.BlockSpec((1, EMB, TN_UP), lambda n, u, m: (m[u], 0, n)),
                pl.BlockSpec((1, EMB, TN_UP), lambda n, u, m: (m[u], 0, n)),
            ],
            out_specs=pl.BlockSpec((TM, TN_UP), lambda n, u, m: (u, n)),
        ),
        out_shape=jax.ShapeDtypeStruct((P_MAX, MLP), jnp.bfloat16),
    )(meta, xs, w0, w1)


def _dn_gmm(meta, inter, wo):
    grid = (EMB // TN_DN, U_MAX)
    return pl.pallas_call(
        _dn_body,
        grid_spec=pltpu.PrefetchScalarGridSpec(
            num_scalar_prefetch=1,
            grid=grid,
            in_specs=[
                pl.BlockSpec((TM, MLP), lambda n, u, m: (u, 0)),
                pl.BlockSpec((1, MLP, TN_DN), lambda n, u, m: (m[u], 0, n)),
            ],
            out_specs=pl.BlockSpec((TM, TN_DN), lambda n, u, m: (u, n)),
        ),
        out_shape=jax.ShapeDtypeStruct((P_MAX, EMB), jnp.float32),
    )(meta, inter, wo)


def kernel(inputs, gate_kernel, w0_kernel, w1_kernel, wo_kernel):
    inputs = inputs.astype(jnp.float32)
    x2 = inputs.reshape(-1, EMB)
    T = x2.shape[0]

    logits = jnp.einsum('bsd,de->bse', inputs, gate_kernel).reshape(T, NUM_EXPERTS)
    a1, a2, wa, wb, r0, r1, counts = _route(logits)

    padded = (counts + TM - 1) // TM * TM
    poff = jnp.concatenate([jnp.zeros((1,), jnp.int32),
                            jnp.cumsum(padded).astype(jnp.int32)])
    pos0 = poff[a1] + r0
    pos1 = poff[a2] + r1
    pos = jnp.stack([pos0, pos1], axis=1).reshape(-1)

    src = jnp.zeros((P_MAX,), jnp.int32).at[pos].set(
        jnp.arange(T * TOP_K, dtype=jnp.int32) // TOP_K)
    xs = jnp.take(x2.astype(jnp.bfloat16), src, axis=0)

    block_expert = jnp.clip(
        jnp.searchsorted(poff, jnp.arange(U_MAX, dtype=jnp.int32) * TM,
                         side="right") - 1,
        0, NUM_EXPERTS - 1).astype(jnp.int32)

    inter = _up_gmm(block_expert, xs, w0_kernel, w1_kernel)
    y = _dn_gmm(block_expert, inter, wo_kernel)

    out = (jnp.take(y, pos0, axis=0) * wa[:, None]
           + jnp.take(y, pos1, axis=0) * wb[:, None])
    return out.reshape(inputs.shape)
